# baseline (speedup 1.0000x reference)
import functools

import jax
import jax.numpy as jnp
from jax import lax
from jax.experimental import pallas as pl
from jax.experimental.pallas import tpu as pltpu
from jax.experimental.pallas import tpu_sc as plsc

_B, _N, _D, _K = 4, 2048, 128, 5
_BH = 2
_NT = 16
_R = _N // _NT
_RB = 256
_NB = _N // _RB
_KP = 8

_HI = lax.Precision.DEFAULT


def _knn_body(xblk_ref, xall_ref, w_ref, idx_ref, dinv_ref, y_ref):
    t = pl.program_id(1)

    @pl.when(t < _NB)
    def _knn_block():
        xb = xblk_ref[0]
        xa = xall_ref[0]
        dot = lax.dot_general(xa, xb, (((1,), (1,)), ((), ())),
                              precision=_HI, preferred_element_type=jnp.float32)
        sqa = jnp.sum(xa * xa, axis=1, keepdims=True)
        ones = jnp.ones((1, _D), jnp.float32)
        sqb = lax.dot_general(ones, xb * xb, (((1,), (1,)), ((), ())),
                              precision=_HI, preferred_element_type=jnp.float32)
        d2 = sqa + sqb - 2.0 * dot
        rowi = lax.broadcasted_iota(jnp.int32, (_N, _RB), 0)
        coli = t * _RB + lax.broadcasted_iota(jnp.int32, (_N, _RB), 1)
        inf = jnp.float32(jnp.inf)
        d2 = jnp.where(rowi == coli, inf, d2)

        S = 8
        sub = lax.broadcasted_iota(jnp.int32, (S, _RB), 0).astype(jnp.float32)
        vals = [jnp.full((S, _RB), inf, jnp.float32) for _ in range(_K)]
        ids = [jnp.full((S, _RB), jnp.float32(_N), jnp.float32)
               for _ in range(_K)]
        for step in range(_N // S):
            v = lax.slice(d2, (step * S, 0), (step * S + S, _RB))
            rid = sub + jnp.float32(step * S)
            c = [v < vals[k] for k in range(_K)]
            for k in reversed(range(_K)):
                if k == 0:
                    vals[0], ids[0] = (jnp.where(c[0], v, vals[0]),
                                       jnp.where(c[0], rid, ids[0]))
                else:
                    vals[k] = jnp.where(c[k],
                                        jnp.where(c[k - 1], vals[k - 1], v),
                                        vals[k])
                    ids[k] = jnp.where(c[k],
                                       jnp.where(c[k - 1], ids[k - 1], rid),
                                       ids[k])

        vs = jnp.concatenate(vals, axis=0)
        vi = jnp.concatenate(ids, axis=0)
        rowf = rowi.astype(jnp.float32)
        macc = jnp.zeros((_N, _RB), jnp.bool_)
        rows = []
        for _ in range(_K):
            m = jnp.min(vs, axis=0, keepdims=True)
            amf = jnp.min(jnp.where(vs == m, vi, jnp.float32(2 * _N)),
                          axis=0, keepdims=True)
            vs = jnp.where(vi == amf, inf, vs)
            macc = macc | (rowf == amf)
            rows.append(amf.astype(jnp.int32))
        deg = jnp.sum(macc.astype(jnp.float32), axis=1, keepdims=True)
        rows.append(jnp.zeros((_KP - _K, _RB), jnp.int32))
        stacked = jnp.concatenate(rows, axis=0)
        for h in range(_RB // _R):
            idx_ref[0, h] = lax.slice(stacked, (0, h * _R), (_KP, (h + 1) * _R))

        @pl.when(t == 0)
        def _():
            dinv_ref[...] = jnp.zeros_like(dinv_ref)

        dinv_ref[...] += deg[None]

        @pl.when(t == _NB - 1)
        def _():
            dinv_ref[...] = lax.rsqrt(dinv_ref[...] + 1.0)

    @pl.when(t == _NB)
    def _lin_step():
        xw = lax.dot_general(xall_ref[0], w_ref[...], (((1,), (0,)), ((), ())),
                             precision=_HI, preferred_element_type=jnp.float32)
        y_ref[0] = xw * dinv_ref[0]


def _lin_body(x_ref, w_ref, dinv_ref, y_ref):
    xw = lax.dot_general(x_ref[0], w_ref[...], (((1,), (0,)), ((), ())),
                         precision=_HI, preferred_element_type=jnp.float32)
    y_ref[0] = xw * dinv_ref[0]


def _knn_lin1(x_half, W1):
    return pl.pallas_call(
        _knn_body,
        grid=(_BH, _NB + 1),
        in_specs=[
            pl.BlockSpec((1, _RB, _D),
                         lambda b, t: (b, jnp.minimum(t, _NB - 1), 0)),
            pl.BlockSpec((1, _N, _D), lambda b, t: (b, 0, 0)),
            pl.BlockSpec((_D, _D), lambda b, t: (0, 0)),
        ],
        out_specs=[
            pl.BlockSpec((1, _RB // _R, _KP, _R),
                         lambda b, t: (b, jnp.minimum(t, _NB - 1), 0, 0)),
            pl.BlockSpec((1, _N, 1), lambda b, t: (b, 0, 0)),
            pl.BlockSpec((1, _N, _D), lambda b, t: (b, 0, 0)),
        ],
        out_shape=[
            jax.ShapeDtypeStruct((_BH, _NT, _KP, _R), jnp.int32),
            jax.ShapeDtypeStruct((_BH, _N, 1), jnp.float32),
            jax.ShapeDtypeStruct((_BH, _N, _D), jnp.float32),
        ],
    )(x_half, x_half, W1)


def _lin(x_half, W, dinv):
    return pl.pallas_call(
        _lin_body,
        grid=(_BH,),
        in_specs=[
            pl.BlockSpec((1, _N, _D), lambda b: (b, 0, 0)),
            pl.BlockSpec((_D, _D), lambda b: (0, 0)),
            pl.BlockSpec((1, _N, 1), lambda b: (b, 0, 0)),
        ],
        out_specs=pl.BlockSpec((1, _N, _D), lambda b: (b, 0, 0)),
        out_shape=jax.ShapeDtypeStruct((_BH, _N, _D), jnp.float32),
    )(x_half, W, dinv)


@functools.lru_cache(maxsize=None)
def _build_scatter(relu):
    mesh = plsc.VectorSubcoreMesh(core_axis_name="c", subcore_axis_name="s")

    @functools.partial(
        pl.kernel,
        out_type=jax.ShapeDtypeStruct((_BH, _N, _D), jnp.float32),
        mesh=mesh,
        scratch_types=[
            pltpu.VMEM((_R, _D), jnp.float32),
            pltpu.VMEM((_KP, _R), jnp.int32),
            pltpu.VMEM((_R,), jnp.float32),
            pltpu.VMEM((_D,), jnp.float32),
            pltpu.VMEM_SHARED((_N, _D), jnp.float32),
        ],
    )
    def scatter(y_hbm, idx_hbm, dinv_hbm, bias_hbm, out_hbm,
                y_v, idx_v, dinv_v, bias_v, acc_sh):
        b = lax.axis_index("c")
        s = lax.axis_index("s")
        base = s * _R
        pltpu.sync_copy(y_hbm.at[b, pl.ds(base, _R)], y_v)
        pltpu.sync_copy(y_v, acc_sh.at[pl.ds(base, _R)])
        pltpu.sync_copy(idx_hbm.at[b, s], idx_v)
        pltpu.sync_copy(dinv_hbm.at[b, pl.ds(base, _R)], dinv_v)
        pltpu.sync_copy(bias_hbm, bias_v)
        plsc.subcore_barrier()
        for k in range(_K):
            pltpu.sync_copy(y_v, acc_sh.at[idx_v.at[k]], add=True)
        plsc.subcore_barrier()
        pltpu.sync_copy(acc_sh.at[pl.ds(base, _R)], y_v)

        def blk(i, carry):
            dvec = dinv_v[pl.ds(i * 16, 16)]
            for rr in range(16):
                sc = dvec[rr]
                r = i * 16 + rr
                for hh in range(_D // 16):
                    seg = (y_v[r, pl.ds(hh * 16, 16)] * sc
                           + bias_v[pl.ds(hh * 16, 16)])
                    if relu:
                        seg = jnp.maximum(seg, jnp.float32(0.0))
                    y_v[r, pl.ds(hh * 16, 16)] = seg
            return carry

        lax.fori_loop(0, _R // 16, blk, 0)
        pltpu.sync_copy(y_v, out_hbm.at[b, pl.ds(base, _R)])

    return scatter


def kernel(x_batch, W1, b1, W2, b2):
    outs = []
    for hf in range(_B // _BH):
        xb = x_batch[hf * _BH:(hf + 1) * _BH]
        idx_t, dinv, y1 = _knn_lin1(xb, W1)
        dinv2 = dinv.reshape(_BH, _N)
        h = _build_scatter(True)(y1, idx_t, dinv2, b1)
        y2 = _lin(h, W2, dinv)
        outs.append(_build_scatter(False)(y2, idx_t, dinv2, b2))
    return jnp.concatenate(outs, axis=0)


# speedup vs baseline: 1.1268x; 1.1268x over previous
import functools

import jax
import jax.numpy as jnp
from jax import lax
from jax.experimental import pallas as pl
from jax.experimental.pallas import tpu as pltpu
from jax.experimental.pallas import tpu_sc as plsc

_B, _N, _D, _K = 4, 2048, 128, 5
_BH = 2
_NT = 16
_R = _N // _NT
_RB = 256
_NB = _N // _RB
_KP = 8

_HI = lax.Precision.DEFAULT


def _knn_body(xblk_ref, xall_ref, w_ref, idx_ref, dinv_ref, y_ref):
    t = pl.program_id(1)

    @pl.when(t < _NB)
    def _knn_block():
        xb = xblk_ref[0]
        xa = xall_ref[0]
        dot = lax.dot_general(xa, xb, (((1,), (1,)), ((), ())),
                              precision=_HI, preferred_element_type=jnp.float32)
        sqa = jnp.sum(xa * xa, axis=1, keepdims=True)
        ones = jnp.ones((1, _D), jnp.float32)
        sqb = lax.dot_general(ones, xb * xb, (((1,), (1,)), ((), ())),
                              precision=_HI, preferred_element_type=jnp.float32)
        d2 = sqa + sqb - 2.0 * dot
        rowi = lax.broadcasted_iota(jnp.int32, (_N, _RB), 0)
        coli = t * _RB + lax.broadcasted_iota(jnp.int32, (_N, _RB), 1)
        inf = jnp.float32(jnp.inf)
        d2 = jnp.where(rowi == coli, inf, d2)

        S = 8
        sub = lax.broadcasted_iota(jnp.int32, (S, _RB), 0).astype(jnp.float32)
        vals = [jnp.full((S, _RB), inf, jnp.float32) for _ in range(_K)]
        ids = [jnp.full((S, _RB), jnp.float32(_N), jnp.float32)
               for _ in range(_K)]
        for step in range(_N // S):
            v = lax.slice(d2, (step * S, 0), (step * S + S, _RB))
            rid = sub + jnp.float32(step * S)
            c = [v < vals[k] for k in range(_K)]
            for k in reversed(range(_K)):
                if k == 0:
                    vals[0], ids[0] = (jnp.where(c[0], v, vals[0]),
                                       jnp.where(c[0], rid, ids[0]))
                else:
                    vals[k] = jnp.where(c[k],
                                        jnp.where(c[k - 1], vals[k - 1], v),
                                        vals[k])
                    ids[k] = jnp.where(c[k],
                                       jnp.where(c[k - 1], ids[k - 1], rid),
                                       ids[k])

        vs = jnp.concatenate(vals, axis=0)
        vi = jnp.concatenate(ids, axis=0)
        rowf = rowi.astype(jnp.float32)
        macc = jnp.zeros((_N, _RB), jnp.bool_)
        rows = []
        for _ in range(_K):
            m = jnp.min(vs, axis=0, keepdims=True)
            amf = jnp.min(jnp.where(vs == m, vi, jnp.float32(2 * _N)),
                          axis=0, keepdims=True)
            vs = jnp.where(vi == amf, inf, vs)
            macc = macc | (rowf == amf)
            rows.append(amf.astype(jnp.int32))
        deg = jnp.sum(macc.astype(jnp.float32), axis=1, keepdims=True)
        rows.append(jnp.zeros((_KP - _K, _RB), jnp.int32))
        stacked = jnp.concatenate(rows, axis=0)
        for h in range(_RB // _R):
            idx_ref[0, h] = lax.slice(stacked, (0, h * _R), (_KP, (h + 1) * _R))

        @pl.when(t == 0)
        def _():
            dinv_ref[...] = jnp.zeros_like(dinv_ref)

        dinv_ref[...] += deg[None]

        @pl.when(t == _NB - 1)
        def _():
            dinv_ref[...] = lax.rsqrt(dinv_ref[...] + 1.0)

    @pl.when(t == _NB)
    def _lin_step():
        xw = lax.dot_general(xall_ref[0], w_ref[...], (((1,), (0,)), ((), ())),
                             precision=_HI, preferred_element_type=jnp.float32)
        y_ref[0] = xw * dinv_ref[0]


def _knn_lin1(x_half, W1):
    return pl.pallas_call(
        _knn_body,
        grid=(_BH, _NB + 1),
        in_specs=[
            pl.BlockSpec((1, _RB, _D),
                         lambda b, t: (b, jnp.minimum(t, _NB - 1), 0)),
            pl.BlockSpec((1, _N, _D), lambda b, t: (b, 0, 0)),
            pl.BlockSpec((_D, _D), lambda b, t: (0, 0)),
        ],
        out_specs=[
            pl.BlockSpec((1, _RB // _R, _KP, _R),
                         lambda b, t: (b, jnp.minimum(t, _NB - 1), 0, 0)),
            pl.BlockSpec((1, _N, 1), lambda b, t: (b, 0, 0)),
            pl.BlockSpec((1, _N, _D), lambda b, t: (b, 0, 0)),
        ],
        out_shape=[
            jax.ShapeDtypeStruct((_BH, _NT, _KP, _R), jnp.int32),
            jax.ShapeDtypeStruct((_BH, _N, 1), jnp.float32),
            jax.ShapeDtypeStruct((_BH, _N, _D), jnp.float32),
        ],
    )(x_half, x_half, W1)


@functools.lru_cache(maxsize=None)
def _build_scatter():
    mesh = plsc.VectorSubcoreMesh(core_axis_name="c", subcore_axis_name="s")

    @functools.partial(
        pl.kernel,
        out_type=jax.ShapeDtypeStruct((_BH, _N, _D), jnp.float32),
        mesh=mesh,
        scratch_types=[
            pltpu.VMEM((_R, _D), jnp.float32),
            pltpu.VMEM((_KP, _R), jnp.int32),
            pltpu.VMEM_SHARED((_N, _D), jnp.float32),
            pltpu.SemaphoreType.DMA,
        ],
    )
    def scatter(y_hbm, idx_hbm, out_hbm, y_v, idx_v, acc_sh, sem):
        b = lax.axis_index("c")
        s = lax.axis_index("s")
        base = s * _R
        h1 = pltpu.async_copy(y_hbm.at[b, pl.ds(base, _R)], y_v, sem)
        h2 = pltpu.async_copy(y_hbm.at[b, pl.ds(base, _R)],
                              acc_sh.at[pl.ds(base, _R)], sem)
        h3 = pltpu.async_copy(idx_hbm.at[b, s], idx_v, sem)
        h1.wait(); h2.wait(); h3.wait()
        plsc.subcore_barrier()
        hs = [pltpu.async_copy(y_v, acc_sh.at[idx_v.at[k]], sem, add=True)
              for k in range(_K)]
        for h in hs:
            h.wait()
        plsc.subcore_barrier()
        pltpu.sync_copy(acc_sh.at[pl.ds(base, _R)],
                        out_hbm.at[b, pl.ds(base, _R)])

    return scatter


def _lin2_body(acc_ref, dinv_ref, b1_ref, w_ref, y_ref):
    h = jnp.maximum(acc_ref[0] * dinv_ref[0] + b1_ref[...], 0.0)
    xw = lax.dot_general(h, w_ref[...], (((1,), (0,)), ((), ())),
                         precision=_HI, preferred_element_type=jnp.float32)
    y_ref[0] = xw * dinv_ref[0]


def _fin_body(acc_ref, dinv_ref, b2_ref, out_ref):
    out_ref[0] = acc_ref[0] * dinv_ref[0] + b2_ref[...]


def _lin2(acc1, dinv, b1r, W2):
    return pl.pallas_call(
        _lin2_body,
        grid=(_BH,),
        in_specs=[
            pl.BlockSpec((1, _N, _D), lambda b: (b, 0, 0)),
            pl.BlockSpec((1, _N, 1), lambda b: (b, 0, 0)),
            pl.BlockSpec((1, _D), lambda b: (0, 0)),
            pl.BlockSpec((_D, _D), lambda b: (0, 0)),
        ],
        out_specs=pl.BlockSpec((1, _N, _D), lambda b: (b, 0, 0)),
        out_shape=jax.ShapeDtypeStruct((_BH, _N, _D), jnp.float32),
    )(acc1, dinv, b1r, W2)


def _fin(acc2, dinv, b2r):
    return pl.pallas_call(
        _fin_body,
        grid=(_BH,),
        in_specs=[
            pl.BlockSpec((1, _N, _D), lambda b: (b, 0, 0)),
            pl.BlockSpec((1, _N, 1), lambda b: (b, 0, 0)),
            pl.BlockSpec((1, _D), lambda b: (0, 0)),
        ],
        out_specs=pl.BlockSpec((1, _N, _D), lambda b: (b, 0, 0)),
        out_shape=jax.ShapeDtypeStruct((_BH, _N, _D), jnp.float32),
    )(acc2, dinv, b2r)


def kernel(x_batch, W1, b1, W2, b2):
    b1r = b1.reshape(1, _D)
    b2r = b2.reshape(1, _D)
    outs = []
    for hf in range(_B // _BH):
        xb = x_batch[hf * _BH:(hf + 1) * _BH]
        idx_t, dinv, y1 = _knn_lin1(xb, W1)
        acc1 = _build_scatter()(y1, idx_t)
        y2 = _lin2(acc1, dinv, b1r, W2)
        acc2 = _build_scatter()(y2, idx_t)
        outs.append(_fin(acc2, dinv, b2r))
    return jnp.concatenate(outs, axis=0)


# speedup vs baseline: 1.1276x; 1.0007x over previous
import functools

import jax
import jax.numpy as jnp
from jax import lax
from jax.experimental import pallas as pl
from jax.experimental.pallas import tpu as pltpu
from jax.experimental.pallas import tpu_sc as plsc

_B, _N, _D, _K = 4, 2048, 128, 5
_BH = 2
_NT = 16
_R = _N // _NT
_RB = 256
_NB = _N // _RB
_KP = 8

_HI = lax.Precision.DEFAULT


def _knn_body(xblk_ref, xall_ref, w_ref, idx_ref, dinv_ref, y_ref):
    t = pl.program_id(1)

    @pl.when(t < _NB)
    def _knn_block():
        xb = xblk_ref[0]
        xa = xall_ref[0]
        dot = lax.dot_general(xa, xb, (((1,), (1,)), ((), ())),
                              precision=_HI, preferred_element_type=jnp.float32)
        sqa = jnp.sum(xa * xa, axis=1, keepdims=True)
        ones = jnp.ones((1, _D), jnp.float32)
        sqb = lax.dot_general(ones, xb * xb, (((1,), (1,)), ((), ())),
                              precision=_HI, preferred_element_type=jnp.float32)
        d2 = sqa + sqb - 2.0 * dot
        rowi = lax.broadcasted_iota(jnp.int32, (_N, _RB), 0)
        coli = t * _RB + lax.broadcasted_iota(jnp.int32, (_N, _RB), 1)
        inf = jnp.float32(jnp.inf)
        d2 = jnp.where(rowi == coli, inf, d2)

        S = 8
        NH = 2
        HROWS = _N // NH
        sub = lax.broadcasted_iota(jnp.int32, (S, _RB), 0).astype(jnp.float32)
        vals = [[jnp.full((S, _RB), inf, jnp.float32) for _ in range(_K)]
                for _ in range(NH)]
        ids = [[jnp.full((S, _RB), jnp.float32(_N), jnp.float32)
                for _ in range(_K)] for _ in range(NH)]
        for step in range(HROWS // S):
            for hh in range(NH):
                r0 = hh * HROWS + step * S
                v = lax.slice(d2, (r0, 0), (r0 + S, _RB))
                rid = sub + jnp.float32(r0)
                va, ia = vals[hh], ids[hh]
                c = [v < va[k] for k in range(_K)]
                for k in reversed(range(_K)):
                    if k == 0:
                        va[0], ia[0] = (jnp.where(c[0], v, va[0]),
                                        jnp.where(c[0], rid, ia[0]))
                    else:
                        va[k] = jnp.where(c[k],
                                          jnp.where(c[k - 1], va[k - 1], v),
                                          va[k])
                        ia[k] = jnp.where(c[k],
                                          jnp.where(c[k - 1], ia[k - 1], rid),
                                          ia[k])

        vs = jnp.concatenate(vals[0] + vals[1], axis=0)
        vi = jnp.concatenate(ids[0] + ids[1], axis=0)
        rowf = rowi.astype(jnp.float32)
        macc = jnp.zeros((_N, _RB), jnp.bool_)
        rows = []
        for _ in range(_K):
            m = jnp.min(vs, axis=0, keepdims=True)
            amf = jnp.min(jnp.where(vs == m, vi, jnp.float32(2 * _N)),
                          axis=0, keepdims=True)
            vs = jnp.where(vi == amf, inf, vs)
            macc = macc | (rowf == amf)
            rows.append(amf.astype(jnp.int32))
        deg = jnp.sum(macc.astype(jnp.float32), axis=1, keepdims=True)
        rows.append(jnp.zeros((_KP - _K, _RB), jnp.int32))
        stacked = jnp.concatenate(rows, axis=0)
        for h in range(_RB // _R):
            idx_ref[0, h] = lax.slice(stacked, (0, h * _R), (_KP, (h + 1) * _R))

        @pl.when(t == 0)
        def _():
            dinv_ref[...] = jnp.zeros_like(dinv_ref)

        dinv_ref[...] += deg[None]

        @pl.when(t == _NB - 1)
        def _():
            dinv_ref[...] = lax.rsqrt(dinv_ref[...] + 1.0)

    @pl.when(t == _NB)
    def _lin_step():
        xw = lax.dot_general(xall_ref[0], w_ref[...], (((1,), (0,)), ((), ())),
                             precision=_HI, preferred_element_type=jnp.float32)
        y_ref[0] = xw * dinv_ref[0]


def _knn_lin1(x_half, W1):
    return pl.pallas_call(
        _knn_body,
        grid=(_BH, _NB + 1),
        in_specs=[
            pl.BlockSpec((1, _RB, _D),
                         lambda b, t: (b, jnp.minimum(t, _NB - 1), 0)),
            pl.BlockSpec((1, _N, _D), lambda b, t: (b, 0, 0)),
            pl.BlockSpec((_D, _D), lambda b, t: (0, 0)),
        ],
        out_specs=[
            pl.BlockSpec((1, _RB // _R, _KP, _R),
                         lambda b, t: (b, jnp.minimum(t, _NB - 1), 0, 0)),
            pl.BlockSpec((1, _N, 1), lambda b, t: (b, 0, 0)),
            pl.BlockSpec((1, _N, _D), lambda b, t: (b, 0, 0)),
        ],
        out_shape=[
            jax.ShapeDtypeStruct((_BH, _NT, _KP, _R), jnp.int32),
            jax.ShapeDtypeStruct((_BH, _N, 1), jnp.float32),
            jax.ShapeDtypeStruct((_BH, _N, _D), jnp.float32),
        ],
    )(x_half, x_half, W1)


@functools.lru_cache(maxsize=None)
def _build_scatter():
    mesh = plsc.VectorSubcoreMesh(core_axis_name="c", subcore_axis_name="s")

    @functools.partial(
        pl.kernel,
        out_type=jax.ShapeDtypeStruct((_BH, _N, _D), jnp.float32),
        mesh=mesh,
        scratch_types=[
            pltpu.VMEM((_R, _D), jnp.float32),
            pltpu.VMEM((_KP, _R), jnp.int32),
            pltpu.VMEM_SHARED((_N, _D), jnp.float32),
            pltpu.SemaphoreType.DMA,
        ],
    )
    def scatter(y_hbm, idx_hbm, out_hbm, y_v, idx_v, acc_sh, sem):
        b = lax.axis_index("c")
        s = lax.axis_index("s")
        base = s * _R
        h1 = pltpu.async_copy(y_hbm.at[b, pl.ds(base, _R)], y_v, sem)
        h2 = pltpu.async_copy(y_hbm.at[b, pl.ds(base, _R)],
                              acc_sh.at[pl.ds(base, _R)], sem)
        h3 = pltpu.async_copy(idx_hbm.at[b, s], idx_v, sem)
        h1.wait(); h2.wait(); h3.wait()
        plsc.subcore_barrier()
        hs = [pltpu.async_copy(y_v, acc_sh.at[idx_v.at[k]], sem, add=True)
              for k in range(_K)]
        for h in hs:
            h.wait()
        plsc.subcore_barrier()
        pltpu.sync_copy(acc_sh.at[pl.ds(base, _R)],
                        out_hbm.at[b, pl.ds(base, _R)])

    return scatter


def _lin2_body(acc_ref, dinv_ref, b1_ref, w_ref, y_ref):
    h = jnp.maximum(acc_ref[0] * dinv_ref[0] + b1_ref[...], 0.0)
    xw = lax.dot_general(h, w_ref[...], (((1,), (0,)), ((), ())),
                         precision=_HI, preferred_element_type=jnp.float32)
    y_ref[0] = xw * dinv_ref[0]


def _fin_body(acc_ref, dinv_ref, b2_ref, out_ref):
    out_ref[0] = acc_ref[0] * dinv_ref[0] + b2_ref[...]


def _lin2(acc1, dinv, b1r, W2):
    return pl.pallas_call(
        _lin2_body,
        grid=(_BH,),
        in_specs=[
            pl.BlockSpec((1, _N, _D), lambda b: (b, 0, 0)),
            pl.BlockSpec((1, _N, 1), lambda b: (b, 0, 0)),
            pl.BlockSpec((1, _D), lambda b: (0, 0)),
            pl.BlockSpec((_D, _D), lambda b: (0, 0)),
        ],
        out_specs=pl.BlockSpec((1, _N, _D), lambda b: (b, 0, 0)),
        out_shape=jax.ShapeDtypeStruct((_BH, _N, _D), jnp.float32),
    )(acc1, dinv, b1r, W2)


def _fin(acc2, dinv, b2r):
    return pl.pallas_call(
        _fin_body,
        grid=(_BH,),
        in_specs=[
            pl.BlockSpec((1, _N, _D), lambda b: (b, 0, 0)),
            pl.BlockSpec((1, _N, 1), lambda b: (b, 0, 0)),
            pl.BlockSpec((1, _D), lambda b: (0, 0)),
        ],
        out_specs=pl.BlockSpec((1, _N, _D), lambda b: (b, 0, 0)),
        out_shape=jax.ShapeDtypeStruct((_BH, _N, _D), jnp.float32),
    )(acc2, dinv, b2r)


def kernel(x_batch, W1, b1, W2, b2):
    b1r = b1.reshape(1, _D)
    b2r = b2.reshape(1, _D)
    outs = []
    for hf in range(_B // _BH):
        xb = x_batch[hf * _BH:(hf + 1) * _BH]
        idx_t, dinv, y1 = _knn_lin1(xb, W1)
        acc1 = _build_scatter()(y1, idx_t)
        y2 = _lin2(acc1, dinv, b1r, W2)
        acc2 = _build_scatter()(y2, idx_t)
        outs.append(_fin(acc2, dinv, b2r))
    return jnp.concatenate(outs, axis=0)
